# baseline (speedup 1.0000x reference)
import functools
import jax
import jax.numpy as jnp
from jax import lax
from jax.experimental import pallas as pl
from jax.experimental.pallas import tpu as pltpu
from jax.experimental.pallas import tpu_sc as plsc


def _pe_add_kernel(B, S, D):
    info = plsc.get_sparse_core_info()
    NC, NS, L = info.num_cores, info.num_subcores, info.num_lanes
    NW = NC * NS
    assert B % NW == 0 and D % L == 0
    seqs_per_w = B // NW
    NSEQ = 4
    assert seqs_per_w % NSEQ == 0
    n_chunks = seqs_per_w // NSEQ

    mesh = plsc.VectorSubcoreMesh(core_axis_name="c", subcore_axis_name="s")

    @functools.partial(
        pl.kernel,
        out_type=jax.ShapeDtypeStruct((B, S, D), jnp.float32),
        mesh=mesh,
        scratch_types=[
            pltpu.VMEM((S, D), jnp.float32),
            pltpu.VMEM((NSEQ, S, D), jnp.float32),
        ],
    )
    def _k(x_hbm, pe_hbm, out_hbm, pe_v, buf):
        wid = lax.axis_index("s") * NC + lax.axis_index("c")
        pltpu.sync_copy(pe_hbm.at[pl.ds(0, S)], pe_v)
        base = wid * seqs_per_w

        @pl.loop(0, n_chunks)
        def _chunk(ci):
            off = base + ci * NSEQ
            pltpu.sync_copy(x_hbm.at[pl.ds(off, NSEQ)], buf)

            @pl.loop(0, S)
            def _row(s):
                for rep in range(NSEQ):
                    for q in range(D // L):
                        sl = pl.ds(q * L, L)
                        buf[rep, s, sl] = buf[rep, s, sl] + pe_v[s, sl]

            pltpu.sync_copy(buf, out_hbm.at[pl.ds(off, NSEQ)])

    return _k


def kernel(x, pe_weight):
    B, S, D = x.shape
    return _pe_add_kernel(B, S, D)(x, pe_weight)
